# Initial kernel scaffold; baseline (speedup 1.0000x reference)
import functools

import jax
import jax.numpy as jnp
from jax import lax
from jax.experimental import pallas as pl
from jax.experimental.pallas import tpu as pltpu
from jax.experimental.pallas import tpu_sc as plsc

B = 4
N = 4096
T = 1024
NH = 16
L = 16
NC = 2
NS = 16
NW = NC * NS
QPW = (B * T) // NW
NGROUPS = QPW // L
CH = 256
NCHUNK = N // CH
KCAP = 512
INF = jnp.float32(jnp.inf)


def _sc_body(x_hbm, coords_hbm, q_hbm, sp_hbm,
             xtab, cbuf, d2buf, bufd, bufn, qbuf, sbuf, sem0, sem1):
    wid = lax.axis_index("s") * NC + lax.axis_index("c")
    b = wid // 8
    q0 = (wid % 8) * QPW

    pltpu.sync_copy(x_hbm.at[b], xtab)

    lanes = lax.iota(jnp.int32, L)

    def group_body(g, sacc):
        t0 = q0 + g * L

        inf_vec = jnp.full((L,), INF, dtype=jnp.float32)

        def clear_body(i, carry):
            r = i // (KCAP // L)
            cidx = i % (KCAP // L)
            bufd[r, pl.ds(cidx * L, L)] = inf_vec
            return carry

        lax.fori_loop(0, L * (KCAP // L), clear_body, 0)

        sems = (sem0, sem1)

        def start_chunk(c, par):
            n0 = c * CH
            h0 = pltpu.async_copy(
                coords_hbm.at[0, b, pl.ds(n0, CH), pl.ds(t0, L)],
                cbuf.at[0, par], sems[par])
            h1 = pltpu.async_copy(
                coords_hbm.at[1, b, pl.ds(n0, CH), pl.ds(t0, L)],
                cbuf.at[1, par], sems[par])
            return (h0, h1)

        stripes = [inf_vec] * L
        pending = start_chunk(0, 0)
        for c in range(NCHUNK):
            par = c % 2
            nxt = None
            if c + 1 < NCHUNK:
                nxt = start_chunk(c + 1, (c + 1) % 2)
            pending[0].wait()
            pending[1].wait()

            n0 = c * CH

            def row_block(nb, carry):
                st = list(carry)
                for j in range(L):
                    row = nb * L + j
                    c0 = cbuf[0, par, row, :]
                    c1 = cbuf[1, par, row, :]
                    d2 = c0 * c0 + c1 * c1
                    d2buf[n0 + row, :] = d2
                    st[j] = jnp.minimum(st[j], d2)
                return tuple(st)

            stripes = lax.fori_loop(0, CH // L, row_block, tuple(stripes))
            stripes = list(stripes)
            if nxt is not None:
                pending = nxt

        thresh = stripes[0]
        for j in range(1, L):
            thresh = jnp.maximum(thresh, stripes[j])

        def scan_body(i, cnt):
            d2 = d2buf[i, :]
            m = d2 <= thresh
            pos = jnp.minimum(cnt, KCAP - 1)
            plsc.store_scatter(bufd, [lanes, pos], d2, mask=m)
            plsc.store_scatter(bufn, [lanes, pos],
                               jnp.full((L,), i, dtype=jnp.int32), mask=m)
            return cnt + jnp.where(m, 1, 0).astype(jnp.int32)

        cnt = lax.fori_loop(0, N, scan_body,
                            jnp.zeros((L,), jnp.int32))

        cntmax = jnp.max(cnt)
        nch = jnp.minimum((cntmax + L - 1) // L, KCAP // L)

        for l in range(L):
            bk0 = bufd[l, pl.ds(0, L)]
            bv0 = bufn[l, pl.ds(0, L)]
            bk, bv = plsc.sort_key_val(bk0, bv0)

            def merge_body(j, carry):
                mk, mv = carry
                ck = bufd[l, pl.ds(j * L, L)]
                cv = bufn[l, pl.ds(j * L, L)]
                ck, cv = plsc.sort_key_val(ck, cv)
                rk = lax.rev(ck, (0,))
                rv = lax.rev(cv, (0,))
                keep = mk <= rk
                nk = jnp.where(keep, mk, rk)
                nv = jnp.where(keep, mv, rv)
                return plsc.sort_key_val(nk, nv)

            bk, bv = lax.fori_loop(1, nch, merge_body, (bk, bv))

            idx = jnp.clip(bv, 0, N - 1)
            xg = plsc.load_gather(xtab, [idx])
            w = jnp.float32(1.0) / jnp.maximum(bk, jnp.float32(1e-30))
            sacc = sacc + w
            qbuf[g * L + l, :] = xg * w
        return sacc

    sacc = lax.fori_loop(0, NGROUPS, group_body,
                         jnp.zeros((L,), jnp.float32))

    sbuf[:] = sacc
    pltpu.sync_copy(qbuf, q_hbm.at[b, pl.ds(q0, QPW)])
    pltpu.sync_copy(sbuf, sp_hbm.at[wid])


def _sc_topk(x2, coords):
    mesh = plsc.VectorSubcoreMesh(core_axis_name="c", subcore_axis_name="s")
    fn = functools.partial(
        pl.kernel, mesh=mesh,
        out_type=(
            jax.ShapeDtypeStruct((B, T, NH), jnp.float32),
            jax.ShapeDtypeStruct((NW, NH), jnp.float32),
        ),
        scratch_types=[
            pltpu.VMEM((N,), jnp.float32),
            pltpu.VMEM((2, 2, CH, L), jnp.float32),
            pltpu.VMEM((N, L), jnp.float32),
            pltpu.VMEM((L, KCAP), jnp.float32),
            pltpu.VMEM((L, KCAP), jnp.int32),
            pltpu.VMEM((QPW, NH), jnp.float32),
            pltpu.VMEM((NH,), jnp.float32),
            pltpu.SemaphoreType.DMA,
            pltpu.SemaphoreType.DMA,
        ],
    )(_sc_body)
    return fn(x2, coords)


def _combine_body(q_ref, sp_ref, out_ref):
    sp = sp_ref[...]
    s = sp.reshape(B, 8, NH).sum(axis=1)
    srows = jnp.tile(s, (16, 1))
    r = (jnp.float32(1.0) / srows).reshape(T)
    for bb in range(B):
        z = q_ref[bb] * r[:, None]
        out_ref[bb] = z.reshape(64, NH, NH).sum(axis=1)


def _combine(q, sp):
    return pl.pallas_call(
        _combine_body,
        out_shape=jax.ShapeDtypeStruct((B, 64, NH), jnp.float32),
    )(q, sp)


def kernel(x, coords_rel):
    x2 = x.reshape(B, N)
    q, sp = _sc_topk(x2, coords_rel)
    r = _combine(q, sp)
    return r.transpose(0, 2, 1).reshape(B, T, 1)


# speedup vs baseline: 7.8144x; 7.8144x over previous
import functools

import jax
import jax.numpy as jnp
from jax import lax
from jax.experimental import pallas as pl
from jax.experimental.pallas import tpu as pltpu
from jax.experimental.pallas import tpu_sc as plsc

B = 4
N = 4096
T = 1024
NH = 16
L = 16
NC = 2
NS = 16
NW = NC * NS
QPW = (B * T) // NW
NGROUPS = QPW // L
CH = 256
NCHUNK = N // CH
KCAP = 512
INF = float("inf")


def _sc_body(x_hbm, coords_hbm, q_hbm, sp_hbm,
             xtab, cbuf, d2buf, bufd, bufn, qbuf, sbuf, sem0, sem1):
    wid = lax.axis_index("s") * NC + lax.axis_index("c")
    b = wid // 8
    q0 = (wid % 8) * QPW

    pltpu.sync_copy(x_hbm.at[b], xtab)

    lanes = lax.iota(jnp.int32, L)

    def group_body(g, sacc):
        t0 = q0 + g * L

        inf_vec = jnp.full((L,), INF, dtype=jnp.float32)

        def clear_body(i, carry):
            r = i // (KCAP // L)
            cidx = i % (KCAP // L)
            bufd[r, pl.ds(cidx * L, L)] = inf_vec
            return carry

        lax.fori_loop(0, L * (KCAP // L), clear_body, 0)

        sems = (sem0, sem1)

        def start_chunk(c, par):
            n0 = c * CH
            h0 = pltpu.async_copy(
                coords_hbm.at[0, b, pl.ds(n0, CH), pl.ds(t0, L)],
                cbuf.at[0, par], sems[par])
            h1 = pltpu.async_copy(
                coords_hbm.at[1, b, pl.ds(n0, CH), pl.ds(t0, L)],
                cbuf.at[1, par], sems[par])
            return (h0, h1)

        stripes = [inf_vec] * L
        pending = start_chunk(0, 0)
        for c in range(NCHUNK):
            par = c % 2
            nxt = None
            if c + 1 < NCHUNK:
                nxt = start_chunk(c + 1, (c + 1) % 2)
            pending[0].wait()
            pending[1].wait()

            n0 = c * CH

            def row_block(nb, carry):
                st = list(carry)
                for j in range(L):
                    row = nb * L + j
                    c0 = cbuf[0, par, row, :]
                    c1 = cbuf[1, par, row, :]
                    d2 = c0 * c0 + c1 * c1
                    d2buf[n0 + row, :] = d2
                    st[j] = jnp.minimum(st[j], d2)
                return tuple(st)

            stripes = lax.fori_loop(0, CH // L, row_block, tuple(stripes))
            stripes = list(stripes)
            if nxt is not None:
                pending = nxt

        thresh = stripes[0]
        for j in range(1, L):
            thresh = jnp.maximum(thresh, stripes[j])

        def scan_body(i, cnt):
            d2 = d2buf[i, :]
            m = d2 <= thresh
            pos = jnp.minimum(cnt, KCAP - 1)
            plsc.store_scatter(bufd, [lanes, pos], d2, mask=m)
            plsc.store_scatter(bufn, [lanes, pos],
                               jnp.full((L,), i, dtype=jnp.int32), mask=m)
            return cnt + jnp.where(m, 1, 0).astype(jnp.int32)

        cnt = lax.fori_loop(0, N, scan_body,
                            jnp.zeros((L,), jnp.int32))

        cntmax = jnp.max(cnt)
        nch = jnp.minimum((cntmax + L - 1) // L, KCAP // L)

        for l in range(L):
            bk0 = bufd[l, pl.ds(0, L)]
            bv0 = bufn[l, pl.ds(0, L)]
            bk, bv = plsc.sort_key_val(bk0, bv0)

            def merge_body(j, carry):
                mk, mv = carry
                ck = bufd[l, pl.ds(j * L, L)]
                cv = bufn[l, pl.ds(j * L, L)]
                ck, cv = plsc.sort_key_val(ck, cv)
                rk = lax.rev(ck, (0,))
                rv = lax.rev(cv, (0,))
                keep = mk <= rk
                nk = jnp.where(keep, mk, rk)
                nv = jnp.where(keep, mv, rv)
                sk, sv = plsc.sort_key_val(nk, nv)
                return (sk, sv)

            bk, bv = lax.fori_loop(1, nch, merge_body, (bk, bv))

            idx = jnp.clip(bv, 0, N - 1)
            xg = plsc.load_gather(xtab, [idx])
            w = jnp.float32(1.0) / jnp.maximum(bk, jnp.float32(1e-30))
            sacc = sacc + w
            qbuf[g * L + l, :] = xg * w
        return sacc

    sacc = lax.fori_loop(0, NGROUPS, group_body,
                         jnp.zeros((L,), jnp.float32))

    sbuf[:] = sacc
    pltpu.sync_copy(qbuf, q_hbm.at[b, pl.ds(q0, QPW)])
    pltpu.sync_copy(sbuf, sp_hbm.at[wid])


def _sc_topk(x2, coords):
    mesh = plsc.VectorSubcoreMesh(core_axis_name="c", subcore_axis_name="s")
    fn = functools.partial(
        pl.kernel, mesh=mesh,
        compiler_params=pltpu.CompilerParams(
            use_tc_tiling_on_sc=False, needs_layout_passes=False),
        out_type=(
            jax.ShapeDtypeStruct((B, T, NH), jnp.float32),
            jax.ShapeDtypeStruct((NW, NH), jnp.float32),
        ),
        scratch_types=[
            pltpu.VMEM((N,), jnp.float32),
            pltpu.VMEM((2, 2, CH, L), jnp.float32),
            pltpu.VMEM((N, L), jnp.float32),
            pltpu.VMEM((L, KCAP), jnp.float32),
            pltpu.VMEM((L, KCAP), jnp.int32),
            pltpu.VMEM((QPW, NH), jnp.float32),
            pltpu.VMEM((NH,), jnp.float32),
            pltpu.SemaphoreType.DMA,
            pltpu.SemaphoreType.DMA,
        ],
    )(_sc_body)
    return fn(x2, coords)


def _combine_body(q_ref, sp_ref, out_ref):
    sp = sp_ref[...]
    rows = [jnp.sum(sp[8 * bb:8 * bb + 8, :], axis=0, keepdims=True)
            for bb in range(B)]
    s = jnp.concatenate(rows, axis=0)
    arec = jnp.float32(1.0) / s
    r4 = lax.broadcasted_iota(jnp.int32, (T, B), 0)
    c4 = lax.broadcasted_iota(jnp.int32, (T, B), 1)
    i4 = ((r4 // 16) % 4 == c4).astype(jnp.float32)
    p1 = jnp.dot(i4, arec, precision=jax.lax.Precision.HIGHEST)
    rt = lax.broadcasted_iota(jnp.int32, (T, NH), 0)
    ck = lax.broadcasted_iota(jnp.int32, (T, NH), 1)
    k16 = (rt % NH == ck).astype(jnp.float32)
    wcol = jnp.sum(k16 * p1, axis=1, keepdims=True)
    ru = lax.broadcasted_iota(jnp.int32, (64, T), 0)
    ct = lax.broadcasted_iota(jnp.int32, (64, T), 1)
    e = (ct // NH == ru).astype(jnp.float32)
    for bb in range(B):
        z = q_ref[bb] * wcol
        out_ref[bb] = jnp.dot(e, z, precision=jax.lax.Precision.HIGHEST)


def _combine(q, sp):
    return pl.pallas_call(
        _combine_body,
        out_shape=jax.ShapeDtypeStruct((B, 64, NH), jnp.float32),
    )(q, sp)


def kernel(x, coords_rel):
    x2 = x.reshape(B, N)
    q, sp = _sc_topk(x2, coords_rel)
    r = _combine(q, sp)
    return r.transpose(0, 2, 1).reshape(B, T, 1)


# speedup vs baseline: 8.8307x; 1.1301x over previous
import functools

import jax
import jax.numpy as jnp
from jax import lax
from jax.experimental import pallas as pl
from jax.experimental.pallas import tpu as pltpu
from jax.experimental.pallas import tpu_sc as plsc

B = 4
N = 4096
T = 1024
NH = 16
L = 16
NC = 2
NS = 16
NW = NC * NS
QPW = (B * T) // NW
NG = QPW // L
CH = 128
NCHUNK = N // CH
KCAP = 128
INF = float("inf")


def _oddeven_pairs(n):
    pairs = []
    p = 1
    while p < n:
        k = p
        while k >= 1:
            for j in range(k % p, n - k, 2 * k):
                for i in range(0, min(k, n - j - k)):
                    if (i + j) // (2 * p) == (i + j + k) // (2 * p):
                        pairs.append((i + j, i + j + k))
            k //= 2
        p *= 2
    return pairs

_SORT32 = _oddeven_pairs(32)


def _sc_body(x_hbm, coords_hbm, q_hbm, sp_hbm,
             xtab, cbuf, stripe, bufd, bufn, thr, cntb, qbuf, sbuf,
             sem0, sem1):
    wid = lax.axis_index("s") * NC + lax.axis_index("c")
    b = wid // 8
    q0 = (wid % 8) * QPW

    pltpu.sync_copy(x_hbm, xtab)

    sems = (sem0, sem1)
    inf_vec = jnp.full((L,), INF, dtype=jnp.float32)
    zero_i = jnp.zeros((L,), jnp.int32)
    lanes = lax.iota(jnp.int32, L)

    def clear_body(i, carry):
        bufd[pl.ds(i * L, L)] = inf_vec
        return carry
    lax.fori_loop(0, (QPW * KCAP) // L, clear_body, 0)
    for g in range(NG):
        cntb[pl.ds(g * L, L)] = zero_i

    def start_chunk(c, par):
        n0 = c * CH
        h0 = pltpu.async_copy(
            coords_hbm.at[0, b, pl.ds(n0, CH), pl.ds(q0, QPW)],
            cbuf.at[0, par], sems[par])
        h1 = pltpu.async_copy(
            coords_hbm.at[1, b, pl.ds(n0, CH), pl.ds(q0, QPW)],
            cbuf.at[1, par], sems[par])
        return (h0, h1)

    def wait_chunk(par):
        pltpu.make_async_copy(
            coords_hbm.at[0, b, pl.ds(0, CH), pl.ds(q0, QPW)],
            cbuf.at[0, par], sems[par]).wait()
        pltpu.make_async_copy(
            coords_hbm.at[1, b, pl.ds(0, CH), pl.ds(q0, QPW)],
            cbuf.at[1, par], sems[par]).wait()

    def s1_group(g, c, par):
        col = g * L

        def rows8(nb, carry):
            m0, m1 = carry
            r0 = nb * 8
            for j in range(8):
                c0 = cbuf[0, par, r0 + j, pl.ds(col, L)]
                c1 = cbuf[1, par, r0 + j, pl.ds(col, L)]
                d2 = c0 * c0 + c1 * c1
                if j % 2 == 0:
                    m0 = jnp.minimum(m0, d2)
                else:
                    m1 = jnp.minimum(m1, d2)
            return (m0, m1)

        m0, m1 = lax.fori_loop(0, CH // 8, rows8, (inf_vec, inf_vec))
        stripe[pl.ds((g * NCHUNK + c) * L, L)] = jnp.minimum(m0, m1)

    def s1_chunk(c, par):
        def gbody(g, carry):
            s1_group(g, c, par)
            return carry
        lax.fori_loop(0, NG, gbody, 0)

    start_chunk(0, 0)
    start_chunk(1, 1)

    def s1_pair(cp, carry):
        c0 = cp * 2
        wait_chunk(0)
        s1_chunk(c0, 0)

        @pl.when(c0 + 2 < NCHUNK)
        def _():
            start_chunk(c0 + 2, 0)

        wait_chunk(1)
        s1_chunk(c0 + 1, 1)

        @pl.when(c0 + 3 < NCHUNK)
        def _():
            start_chunk(c0 + 3, 1)
        return carry

    lax.fori_loop(0, NCHUNK // 2, s1_pair, 0)

    def thr_body(g, carry):
        v = [stripe[pl.ds((g * NCHUNK + c) * L, L)] for c in range(NCHUNK)]
        for a, bb in _SORT32:
            lo = jnp.minimum(v[a], v[bb])
            hi = jnp.maximum(v[a], v[bb])
            v[a], v[bb] = lo, hi
        thr[pl.ds(g * L, L)] = v[NH - 1]
        return carry

    lax.fori_loop(0, NG, thr_body, 0)

    rowbase = lanes * KCAP

    def s2_group(g, c, par):
        col = g * L
        thresh = thr[pl.ds(g * L, L)]
        cnt0 = cntb[pl.ds(g * L, L)]
        gbase = rowbase + (g * L) * KCAP
        n0 = c * CH

        def rows4(nb, cnt):
            r0 = nb * 4
            for j in range(4):
                c0 = cbuf[0, par, r0 + j, pl.ds(col, L)]
                c1 = cbuf[1, par, r0 + j, pl.ds(col, L)]
                d2 = c0 * c0 + c1 * c1
                m = d2 <= thresh
                pos = jnp.minimum(cnt, KCAP - 1)
                idxv = gbase + pos
                plsc.store_scatter(bufd, [idxv], d2, mask=m)
                plsc.store_scatter(bufn, [idxv],
                                   jnp.full((L,), n0 + r0 + j, jnp.int32) ,
                                   mask=m)
                cnt = cnt + jnp.where(m, 1, 0).astype(jnp.int32)
            return cnt

        cnt = lax.fori_loop(0, CH // 4, rows4, cnt0)
        cntb[pl.ds(g * L, L)] = cnt

    def s2_chunk(c, par):
        def gbody(g, carry):
            s2_group(g, c, par)
            return carry
        lax.fori_loop(0, NG, gbody, 0)

    start_chunk(0, 0)
    start_chunk(1, 1)

    def s2_pair(cp, carry):
        c0 = cp * 2
        wait_chunk(0)
        s2_chunk(c0, 0)

        @pl.when(c0 + 2 < NCHUNK)
        def _():
            start_chunk(c0 + 2, 0)

        wait_chunk(1)
        s2_chunk(c0 + 1, 1)

        @pl.when(c0 + 3 < NCHUNK)
        def _():
            start_chunk(c0 + 3, 1)
        return carry

    lax.fori_loop(0, NCHUNK // 2, s2_pair, 0)

    xoff = b * N

    def sel_body(g, sacc):
        cnt = cntb[pl.ds(g * L, L)]
        cmax = jnp.minimum(jnp.max(cnt), KCAP)
        nch = (cmax + L - 1) // L
        for l in range(L):
            base = (g * L + l) * KCAP
            bk, bv = plsc.sort_key_val(bufd[pl.ds(base, L)],
                                       bufn[pl.ds(base, L)])

            def merge_body(j, carry):
                mk, mv = carry
                ck, cv = plsc.sort_key_val(bufd[pl.ds(base + j * L, L)],
                                           bufn[pl.ds(base + j * L, L)])
                rk = lax.rev(ck, (0,))
                rv = lax.rev(cv, (0,))
                keep = mk <= rk
                nk = jnp.where(keep, mk, rk)
                nv = jnp.where(keep, mv, rv)
                sk, sv = plsc.sort_key_val(nk, nv)
                return (sk, sv)

            bk, bv = lax.fori_loop(1, nch, merge_body, (bk, bv))

            idx = jnp.clip(bv, 0, N - 1) + xoff
            xg = plsc.load_gather(xtab, [idx])
            w = jnp.float32(1.0) / jnp.maximum(bk, jnp.float32(1e-30))
            sacc = sacc + w
            qbuf[pl.ds((g * L + l) * NH, NH)] = xg * w
        return sacc

    sacc = lax.fori_loop(0, NG, sel_body, jnp.zeros((L,), jnp.float32))

    sbuf[...] = sacc
    pltpu.sync_copy(qbuf, q_hbm.at[pl.ds((b * T + q0) * NH, QPW * NH)])
    pltpu.sync_copy(sbuf, sp_hbm.at[pl.ds(wid * NH, NH)])


def _sc_topk(xflat, coords):
    mesh = plsc.VectorSubcoreMesh(core_axis_name="c", subcore_axis_name="s")
    fn = functools.partial(
        pl.kernel, mesh=mesh,
        compiler_params=pltpu.CompilerParams(needs_layout_passes=False),
        out_type=(
            jax.ShapeDtypeStruct((B * T * NH,), jnp.float32),
            jax.ShapeDtypeStruct((NW * NH,), jnp.float32),
        ),
        scratch_types=[
            pltpu.VMEM((B * N,), jnp.float32),
            pltpu.VMEM((2, 2, CH, QPW), jnp.float32),
            pltpu.VMEM((NG * NCHUNK * L,), jnp.float32),
            pltpu.VMEM((QPW * KCAP,), jnp.float32),
            pltpu.VMEM((QPW * KCAP,), jnp.int32),
            pltpu.VMEM((NG * L,), jnp.float32),
            pltpu.VMEM((NG * L,), jnp.int32),
            pltpu.VMEM((QPW * NH,), jnp.float32),
            pltpu.VMEM((NH,), jnp.float32),
            pltpu.SemaphoreType.DMA,
            pltpu.SemaphoreType.DMA,
        ],
    )(_sc_body)
    return fn(xflat, coords)


def _combine_body(q_ref, sp_ref, out_ref):
    sp = sp_ref[...]
    rows = [jnp.sum(sp[8 * bb:8 * bb + 8, :], axis=0, keepdims=True)
            for bb in range(B)]
    s = jnp.concatenate(rows, axis=0)
    arec = jnp.float32(1.0) / s
    r4 = lax.broadcasted_iota(jnp.int32, (T, B), 0)
    c4 = lax.broadcasted_iota(jnp.int32, (T, B), 1)
    i4 = ((r4 // 16) % 4 == c4).astype(jnp.float32)
    p1 = jnp.dot(i4, arec, precision=jax.lax.Precision.HIGHEST)
    rt = lax.broadcasted_iota(jnp.int32, (T, NH), 0)
    ck = lax.broadcasted_iota(jnp.int32, (T, NH), 1)
    k16 = (rt % NH == ck).astype(jnp.float32)
    wcol = jnp.sum(k16 * p1, axis=1, keepdims=True)
    ru = lax.broadcasted_iota(jnp.int32, (64, T), 0)
    ct = lax.broadcasted_iota(jnp.int32, (64, T), 1)
    e = (ct // NH == ru).astype(jnp.float32)
    for bb in range(B):
        z = q_ref[bb] * wcol
        out_ref[bb] = jnp.dot(e, z, precision=jax.lax.Precision.HIGHEST)


def _combine(q, sp):
    return pl.pallas_call(
        _combine_body,
        out_shape=jax.ShapeDtypeStruct((B, 64, NH), jnp.float32),
    )(q, sp)


def kernel(x, coords_rel):
    xflat = x.reshape(B * N)
    qflat, spflat = _sc_topk(xflat, coords_rel)
    q = qflat.reshape(B, T, NH)
    sp = spflat.reshape(NW, NH)
    r = _combine(q, sp)
    return r.transpose(0, 2, 1).reshape(B, T, 1)


# speedup vs baseline: 8.8409x; 1.0012x over previous
import functools

import jax
import jax.numpy as jnp
from jax import lax
from jax.experimental import pallas as pl
from jax.experimental.pallas import tpu as pltpu
from jax.experimental.pallas import tpu_sc as plsc

B = 4
N = 4096
T = 1024
NH = 16
L = 16
NC = 2
NS = 16
NW = NC * NS
QPW = (B * T) // NW
NG = QPW // L
CH = 128
NCHUNK = N // CH
KCAP = 128
INF = float("inf")


def _oddeven_pairs(n):
    pairs = []
    p = 1
    while p < n:
        k = p
        while k >= 1:
            for j in range(k % p, n - k, 2 * k):
                for i in range(0, min(k, n - j - k)):
                    if (i + j) // (2 * p) == (i + j + k) // (2 * p):
                        pairs.append((i + j, i + j + k))
            k //= 2
        p *= 2
    return pairs

_SORT32 = _oddeven_pairs(32)


def _sc_body(x_hbm, coords_hbm, q_hbm, sp_hbm,
             xtab, cbuf, stripe, bufd, bufn, thr, cntb, qbuf, sbuf,
             sem0, sem1):
    wid = lax.axis_index("s") * NC + lax.axis_index("c")
    b = wid // 8
    q0 = (wid % 8) * QPW

    pltpu.sync_copy(x_hbm, xtab)

    sems = (sem0, sem1)
    inf_vec = jnp.full((L,), INF, dtype=jnp.float32)
    zero_i = jnp.zeros((L,), jnp.int32)
    lanes = lax.iota(jnp.int32, L)

    def clear_body(i, carry):
        bufd[pl.ds(i * L, L)] = inf_vec
        return carry
    lax.fori_loop(0, (QPW * KCAP) // L, clear_body, 0)
    for g in range(NG):
        cntb[pl.ds(g * L, L)] = zero_i

    def start_chunk(c, par):
        n0 = c * CH
        h0 = pltpu.async_copy(
            coords_hbm.at[0, b, pl.ds(n0, CH), pl.ds(q0, QPW)],
            cbuf.at[0, par], sems[par])
        h1 = pltpu.async_copy(
            coords_hbm.at[1, b, pl.ds(n0, CH), pl.ds(q0, QPW)],
            cbuf.at[1, par], sems[par])
        return (h0, h1)

    def wait_chunk(par):
        pltpu.make_async_copy(
            coords_hbm.at[0, b, pl.ds(0, CH), pl.ds(q0, QPW)],
            cbuf.at[0, par], sems[par]).wait()
        pltpu.make_async_copy(
            coords_hbm.at[1, b, pl.ds(0, CH), pl.ds(q0, QPW)],
            cbuf.at[1, par], sems[par]).wait()

    def s1_group(g, c, par):
        col = g * L

        def rows8(nb, carry):
            m0, m1 = carry
            r0 = nb * 8
            for j in range(8):
                c0 = cbuf[0, par, r0 + j, pl.ds(col, L)]
                c1 = cbuf[1, par, r0 + j, pl.ds(col, L)]
                d2 = c0 * c0 + c1 * c1
                if j % 2 == 0:
                    m0 = jnp.minimum(m0, d2)
                else:
                    m1 = jnp.minimum(m1, d2)
            return (m0, m1)

        m0, m1 = lax.fori_loop(0, CH // 8, rows8, (inf_vec, inf_vec))
        stripe[pl.ds((g * NCHUNK + c) * L, L)] = jnp.minimum(m0, m1)

    def s1_chunk(c, par):
        def gbody(g, carry):
            s1_group(g, c, par)
            return carry
        lax.fori_loop(0, NG, gbody, 0)

    start_chunk(0, 0)
    start_chunk(1, 1)

    def s1_pair(cp, carry):
        c0 = cp * 2
        wait_chunk(0)
        s1_chunk(c0, 0)

        @pl.when(c0 + 2 < NCHUNK)
        def _():
            start_chunk(c0 + 2, 0)

        wait_chunk(1)
        s1_chunk(c0 + 1, 1)

        @pl.when(c0 + 3 < NCHUNK)
        def _():
            start_chunk(c0 + 3, 1)
        return carry

    with jax.named_scope("s1"):
        lax.fori_loop(0, NCHUNK // 2, s1_pair, 0)

    def thr_body(g, carry):
        v = [stripe[pl.ds((g * NCHUNK + c) * L, L)] for c in range(NCHUNK)]
        for a, bb in _SORT32:
            lo = jnp.minimum(v[a], v[bb])
            hi = jnp.maximum(v[a], v[bb])
            v[a], v[bb] = lo, hi
        thr[pl.ds(g * L, L)] = v[NH - 1]
        return carry

    with jax.named_scope("thr"):
        lax.fori_loop(0, NG, thr_body, 0)

    rowbase = lanes * KCAP

    def s2_group(g, c, par):
        col = g * L
        thresh = thr[pl.ds(g * L, L)]
        cnt0 = cntb[pl.ds(g * L, L)]
        gbase = rowbase + (g * L) * KCAP
        n0 = c * CH

        def rows4(nb, cnt):
            r0 = nb * 4
            for j in range(4):
                c0 = cbuf[0, par, r0 + j, pl.ds(col, L)]
                c1 = cbuf[1, par, r0 + j, pl.ds(col, L)]
                d2 = c0 * c0 + c1 * c1
                m = d2 <= thresh
                pos = jnp.minimum(cnt, KCAP - 1)
                idxv = gbase + pos
                plsc.store_scatter(bufd, [idxv], d2, mask=m)
                plsc.store_scatter(bufn, [idxv],
                                   jnp.full((L,), n0 + r0 + j, jnp.int32) ,
                                   mask=m)
                cnt = cnt + jnp.where(m, 1, 0).astype(jnp.int32)
            return cnt

        cnt = lax.fori_loop(0, CH // 4, rows4, cnt0)
        cntb[pl.ds(g * L, L)] = cnt

    def s2_chunk(c, par):
        def gbody(g, carry):
            s2_group(g, c, par)
            return carry
        lax.fori_loop(0, NG, gbody, 0)

    start_chunk(0, 0)
    start_chunk(1, 1)

    def s2_pair(cp, carry):
        c0 = cp * 2
        wait_chunk(0)
        s2_chunk(c0, 0)

        @pl.when(c0 + 2 < NCHUNK)
        def _():
            start_chunk(c0 + 2, 0)

        wait_chunk(1)
        s2_chunk(c0 + 1, 1)

        @pl.when(c0 + 3 < NCHUNK)
        def _():
            start_chunk(c0 + 3, 1)
        return carry

    with jax.named_scope("s2"):
        lax.fori_loop(0, NCHUNK // 2, s2_pair, 0)

    xoff = b * N

    def sel_body(g, sacc):
        cnt = cntb[pl.ds(g * L, L)]
        cmax = jnp.minimum(jnp.max(cnt), KCAP)
        nch = (cmax + L - 1) // L
        for l in range(L):
            base = (g * L + l) * KCAP
            bk, bv = plsc.sort_key_val(bufd[pl.ds(base, L)],
                                       bufn[pl.ds(base, L)])

            def merge_body(j, carry):
                mk, mv = carry
                ck, cv = plsc.sort_key_val(bufd[pl.ds(base + j * L, L)],
                                           bufn[pl.ds(base + j * L, L)])
                rk = lax.rev(ck, (0,))
                rv = lax.rev(cv, (0,))
                keep = mk <= rk
                nk = jnp.where(keep, mk, rk)
                nv = jnp.where(keep, mv, rv)
                sk, sv = plsc.sort_key_val(nk, nv)
                return (sk, sv)

            bk, bv = lax.fori_loop(1, nch, merge_body, (bk, bv))

            idx = jnp.clip(bv, 0, N - 1) + xoff
            xg = plsc.load_gather(xtab, [idx])
            w = jnp.float32(1.0) / jnp.maximum(bk, jnp.float32(1e-30))
            sacc = sacc + w
            qbuf[pl.ds((g * L + l) * NH, NH)] = xg * w
        return sacc

    with jax.named_scope("sel"):
        sacc = lax.fori_loop(0, NG, sel_body, jnp.zeros((L,), jnp.float32))

    sbuf[...] = sacc
    pltpu.sync_copy(qbuf, q_hbm.at[pl.ds((b * T + q0) * NH, QPW * NH)])
    pltpu.sync_copy(sbuf, sp_hbm.at[pl.ds(wid * NH, NH)])


def _sc_topk(xflat, coords):
    mesh = plsc.VectorSubcoreMesh(core_axis_name="c", subcore_axis_name="s")
    fn = functools.partial(
        pl.kernel, mesh=mesh,
        compiler_params=pltpu.CompilerParams(needs_layout_passes=False),
        out_type=(
            jax.ShapeDtypeStruct((B * T * NH,), jnp.float32),
            jax.ShapeDtypeStruct((NW * NH,), jnp.float32),
        ),
        scratch_types=[
            pltpu.VMEM((B * N,), jnp.float32),
            pltpu.VMEM((2, 2, CH, QPW), jnp.float32),
            pltpu.VMEM((NG * NCHUNK * L,), jnp.float32),
            pltpu.VMEM((QPW * KCAP,), jnp.float32),
            pltpu.VMEM((QPW * KCAP,), jnp.int32),
            pltpu.VMEM((NG * L,), jnp.float32),
            pltpu.VMEM((NG * L,), jnp.int32),
            pltpu.VMEM((QPW * NH,), jnp.float32),
            pltpu.VMEM((NH,), jnp.float32),
            pltpu.SemaphoreType.DMA,
            pltpu.SemaphoreType.DMA,
        ],
    )(_sc_body)
    return fn(xflat, coords)


def _combine_body(q_ref, sp_ref, out_ref):
    sp = sp_ref[...]
    rows = [jnp.sum(sp[8 * bb:8 * bb + 8, :], axis=0, keepdims=True)
            for bb in range(B)]
    s = jnp.concatenate(rows, axis=0)
    arec = jnp.float32(1.0) / s
    r4 = lax.broadcasted_iota(jnp.int32, (T, B), 0)
    c4 = lax.broadcasted_iota(jnp.int32, (T, B), 1)
    i4 = ((r4 // 16) % 4 == c4).astype(jnp.float32)
    p1 = jnp.dot(i4, arec, precision=jax.lax.Precision.HIGHEST)
    rt = lax.broadcasted_iota(jnp.int32, (T, NH), 0)
    ck = lax.broadcasted_iota(jnp.int32, (T, NH), 1)
    k16 = (rt % NH == ck).astype(jnp.float32)
    wcol = jnp.sum(k16 * p1, axis=1, keepdims=True)
    ru = lax.broadcasted_iota(jnp.int32, (64, T), 0)
    ct = lax.broadcasted_iota(jnp.int32, (64, T), 1)
    e = (ct // NH == ru).astype(jnp.float32)
    for bb in range(B):
        z = q_ref[bb] * wcol
        out_ref[bb] = jnp.dot(e, z, precision=jax.lax.Precision.HIGHEST)


def _combine(q, sp):
    return pl.pallas_call(
        _combine_body,
        out_shape=jax.ShapeDtypeStruct((B, 64, NH), jnp.float32),
    )(q, sp)


def kernel(x, coords_rel):
    xflat = x.reshape(B * N)
    qflat, spflat = _sc_topk(xflat, coords_rel)
    q = qflat.reshape(B, T, NH)
    sp = spflat.reshape(NW, NH)
    r = _combine(q, sp)
    return r.transpose(0, 2, 1).reshape(B, T, 1)


# speedup vs baseline: 9.9490x; 1.1253x over previous
import functools

import jax
import jax.numpy as jnp
from jax import lax
from jax.experimental import pallas as pl
from jax.experimental.pallas import tpu as pltpu
from jax.experimental.pallas import tpu_sc as plsc

B = 4
N = 4096
T = 1024
NH = 16
L = 16
NC = 2
NS = 16
NW = NC * NS
QPW = (B * T) // NW
NG = QPW // L
CH = 128
NCHUNK = N // CH
KCAP = 128
INF = float("inf")


def _oddeven_pairs(n):
    pairs = []
    p = 1
    while p < n:
        k = p
        while k >= 1:
            for j in range(k % p, n - k, 2 * k):
                for i in range(0, min(k, n - j - k)):
                    if (i + j) // (2 * p) == (i + j + k) // (2 * p):
                        pairs.append((i + j, i + j + k))
            k //= 2
        p *= 2
    return pairs

_SORT32 = _oddeven_pairs(32)


def _sc_body(x_hbm, d2_hbm, m_hbm, q_hbm, sp_hbm,
             xtab, cbuf, mbuf, bufd, bufn, thr, cntb, qbuf, sbuf,
             sem0, sem1):
    wid = lax.axis_index("s") * NC + lax.axis_index("c")
    b = wid // 8
    q0 = (wid % 8) * QPW

    pltpu.sync_copy(x_hbm, xtab)
    pltpu.sync_copy(m_hbm.at[b, pl.ds(0, NCHUNK), pl.ds(q0, QPW)], mbuf)

    sems = (sem0, sem1)
    inf_vec = jnp.full((L,), INF, dtype=jnp.float32)
    zero_i = jnp.zeros((L,), jnp.int32)
    lanes = lax.iota(jnp.int32, L)

    def clear_body(i, carry):
        bufd[pl.ds(i * L, L)] = inf_vec
        return carry
    lax.fori_loop(0, (QPW * KCAP) // L, clear_body, 0)
    for g in range(NG):
        cntb[pl.ds(g * L, L)] = zero_i

    def start_chunk(c, par):
        n0 = c * CH
        return pltpu.async_copy(
            d2_hbm.at[b, pl.ds(n0, CH), pl.ds(q0, QPW)],
            cbuf.at[par], sems[par])

    def wait_chunk(par):
        pltpu.make_async_copy(
            d2_hbm.at[b, pl.ds(0, CH), pl.ds(q0, QPW)],
            cbuf.at[par], sems[par]).wait()

    def thr_body(g, carry):
        v = [mbuf[c, pl.ds(g * L, L)] for c in range(NCHUNK)]
        for a, bb in _SORT32:
            lo = jnp.minimum(v[a], v[bb])
            hi = jnp.maximum(v[a], v[bb])
            v[a], v[bb] = lo, hi
        thr[pl.ds(g * L, L)] = v[NH - 1]
        return carry

    with jax.named_scope("thr"):
        lax.fori_loop(0, NG, thr_body, 0)

    rowbase = lanes * KCAP

    def s2_group(g, c, par):
        col = g * L
        thresh = thr[pl.ds(g * L, L)]
        cnt0 = cntb[pl.ds(g * L, L)]
        gbase = rowbase + (g * L) * KCAP
        n0 = c * CH

        def rows4(nb, cnt):
            r0 = nb * 4
            for j in range(4):
                d2 = cbuf[par, r0 + j, pl.ds(col, L)]
                m = d2 <= thresh
                pos = jnp.minimum(cnt, KCAP - 1)
                idxv = gbase + pos
                plsc.store_scatter(bufd, [idxv], d2, mask=m)
                plsc.store_scatter(bufn, [idxv],
                                   jnp.full((L,), n0 + r0 + j, jnp.int32) ,
                                   mask=m)
                cnt = cnt + jnp.where(m, 1, 0).astype(jnp.int32)
            return cnt

        cnt = lax.fori_loop(0, CH // 4, rows4, cnt0)
        cntb[pl.ds(g * L, L)] = cnt

    def s2_chunk(c, par):
        def gbody(g, carry):
            s2_group(g, c, par)
            return carry
        lax.fori_loop(0, NG, gbody, 0)

    start_chunk(0, 0)
    start_chunk(1, 1)

    def s2_pair(cp, carry):
        c0 = cp * 2
        wait_chunk(0)
        s2_chunk(c0, 0)

        @pl.when(c0 + 2 < NCHUNK)
        def _():
            start_chunk(c0 + 2, 0)

        wait_chunk(1)
        s2_chunk(c0 + 1, 1)

        @pl.when(c0 + 3 < NCHUNK)
        def _():
            start_chunk(c0 + 3, 1)
        return carry

    with jax.named_scope("s2"):
        lax.fori_loop(0, NCHUNK // 2, s2_pair, 0)

    xoff = b * N

    def sel_body(g, sacc):
        cnt = cntb[pl.ds(g * L, L)]
        cmax = jnp.minimum(jnp.max(cnt), KCAP)
        nch = (cmax + L - 1) // L
        for l in range(L):
            base = (g * L + l) * KCAP
            bk, bv = plsc.sort_key_val(bufd[pl.ds(base, L)],
                                       bufn[pl.ds(base, L)])

            def merge_body(j, carry):
                mk, mv = carry
                ck, cv = plsc.sort_key_val(bufd[pl.ds(base + j * L, L)],
                                           bufn[pl.ds(base + j * L, L)])
                rk = lax.rev(ck, (0,))
                rv = lax.rev(cv, (0,))
                keep = mk <= rk
                nk = jnp.where(keep, mk, rk)
                nv = jnp.where(keep, mv, rv)
                sk, sv = plsc.sort_key_val(nk, nv)
                return (sk, sv)

            bk, bv = lax.fori_loop(1, nch, merge_body, (bk, bv))

            idx = jnp.clip(bv, 0, N - 1) + xoff
            xg = plsc.load_gather(xtab, [idx])
            w = jnp.float32(1.0) / jnp.maximum(bk, jnp.float32(1e-30))
            sacc = sacc + w
            qbuf[pl.ds((g * L + l) * NH, NH)] = xg * w
        return sacc

    with jax.named_scope("sel"):
        sacc = lax.fori_loop(0, NG, sel_body, jnp.zeros((L,), jnp.float32))

    sbuf[...] = sacc
    pltpu.sync_copy(qbuf, q_hbm.at[pl.ds((b * T + q0) * NH, QPW * NH)])
    pltpu.sync_copy(sbuf, sp_hbm.at[pl.ds(wid * NH, NH)])


def _sc_topk(xflat, d2, m):
    mesh = plsc.VectorSubcoreMesh(core_axis_name="c", subcore_axis_name="s")
    fn = functools.partial(
        pl.kernel, mesh=mesh,
        compiler_params=pltpu.CompilerParams(needs_layout_passes=False),
        out_type=(
            jax.ShapeDtypeStruct((B * T * NH,), jnp.float32),
            jax.ShapeDtypeStruct((NW * NH,), jnp.float32),
        ),
        scratch_types=[
            pltpu.VMEM((B * N,), jnp.float32),
            pltpu.VMEM((2, CH, QPW), jnp.float32),
            pltpu.VMEM((NCHUNK, QPW), jnp.float32),
            pltpu.VMEM((QPW * KCAP,), jnp.float32),
            pltpu.VMEM((QPW * KCAP,), jnp.int32),
            pltpu.VMEM((NG * L,), jnp.float32),
            pltpu.VMEM((NG * L,), jnp.int32),
            pltpu.VMEM((QPW * NH,), jnp.float32),
            pltpu.VMEM((NH,), jnp.float32),
            pltpu.SemaphoreType.DMA,
            pltpu.SemaphoreType.DMA,
        ],
    )(_sc_body)
    return fn(xflat, d2, m)


def _d2_body(coords_ref, d2_ref, m_ref):
    c0 = coords_ref[0, 0]
    c1 = coords_ref[1, 0]
    d2 = c0 * c0 + c1 * c1
    d2_ref[0] = d2
    m_ref[0, pl.program_id(1)] = jnp.min(d2, axis=0)


def _tc_d2(coords):
    return pl.pallas_call(
        _d2_body,
        grid=(B, NCHUNK),
        in_specs=[pl.BlockSpec((2, 1, CH, T), lambda b, c: (0, b, c, 0))],
        out_specs=[
            pl.BlockSpec((1, CH, T), lambda b, c: (b, c, 0)),
            pl.BlockSpec((1, NCHUNK, T), lambda b, c: (b, 0, 0)),
        ],
        out_shape=[
            jax.ShapeDtypeStruct((B, N, T), jnp.float32),
            jax.ShapeDtypeStruct((B, NCHUNK, T), jnp.float32),
        ],
    )(coords)


def _combine_body(q_ref, sp_ref, out_ref):
    sp = sp_ref[...]
    rows = [jnp.sum(sp[8 * bb:8 * bb + 8, :], axis=0, keepdims=True)
            for bb in range(B)]
    s = jnp.concatenate(rows, axis=0)
    arec = jnp.float32(1.0) / s
    r4 = lax.broadcasted_iota(jnp.int32, (T, B), 0)
    c4 = lax.broadcasted_iota(jnp.int32, (T, B), 1)
    i4 = ((r4 // 16) % 4 == c4).astype(jnp.float32)
    p1 = jnp.dot(i4, arec, precision=jax.lax.Precision.HIGHEST)
    rt = lax.broadcasted_iota(jnp.int32, (T, NH), 0)
    ck = lax.broadcasted_iota(jnp.int32, (T, NH), 1)
    k16 = (rt % NH == ck).astype(jnp.float32)
    wcol = jnp.sum(k16 * p1, axis=1, keepdims=True)
    ru = lax.broadcasted_iota(jnp.int32, (64, T), 0)
    ct = lax.broadcasted_iota(jnp.int32, (64, T), 1)
    e = (ct // NH == ru).astype(jnp.float32)
    for bb in range(B):
        z = q_ref[bb] * wcol
        out_ref[bb] = jnp.dot(e, z, precision=jax.lax.Precision.HIGHEST)


def _combine(q, sp):
    return pl.pallas_call(
        _combine_body,
        out_shape=jax.ShapeDtypeStruct((B, 64, NH), jnp.float32),
    )(q, sp)


def kernel(x, coords_rel):
    xflat = x.reshape(B * N)
    d2, m = _tc_d2(coords_rel)
    qflat, spflat = _sc_topk(xflat, d2, m)
    q = qflat.reshape(B, T, NH)
    sp = spflat.reshape(NW, NH)
    r = _combine(q, sp)
    return r.transpose(0, 2, 1).reshape(B, T, 1)


# speedup vs baseline: 15.6810x; 1.5761x over previous
import functools

import jax
import jax.numpy as jnp
from jax import lax
from jax.experimental import pallas as pl
from jax.experimental.pallas import tpu as pltpu
from jax.experimental.pallas import tpu_sc as plsc

B = 4
N = 4096
T = 1024
NH = 16
L = 16
NC = 2
NS = 16
NW = NC * NS
QPW = (B * T) // NW
BLK = 16
NB = N // BLK
NBV = NB // L
BT = 128
SCAP = N + L
INF = float("inf")


def _d2t_body(coords_ref, d2t_ref, bm_ref):
    c0 = coords_ref[0, 0]
    c1 = coords_ref[1, 0]
    d2 = c0 * c0 + c1 * c1
    bm = jnp.min(d2.reshape(NB, BLK, BT), axis=1)
    d2t_ref[0] = d2.T
    bm_ref[0] = bm.T


def _tc_d2t(coords):
    return pl.pallas_call(
        _d2t_body,
        grid=(B, T // BT),
        in_specs=[pl.BlockSpec((2, 1, N, BT), lambda b, t: (0, b, 0, t))],
        out_specs=[
            pl.BlockSpec((1, BT, N), lambda b, t: (b, t, 0)),
            pl.BlockSpec((1, BT, NB), lambda b, t: (b, t, 0)),
        ],
        out_shape=[
            jax.ShapeDtypeStruct((B, T, N), jnp.float32),
            jax.ShapeDtypeStruct((B, T, NB), jnp.float32),
        ],
    )(coords)


def _sc_body(x_hbm, d2t_hbm, bm_hbm, q_hbm, sp_hbm,
             xb, rowring, bmall, blkb, bufd, bufn, qbuf, sbuf, sem0, sem1):
    wid = lax.axis_index("s") * NC + lax.axis_index("c")
    b = wid // 8
    q0 = (wid % 8) * QPW

    pltpu.sync_copy(x_hbm.at[pl.ds(b * N, N)], xb)
    pltpu.sync_copy(bm_hbm.at[b, pl.ds(q0, QPW), :], bmall)

    sems = (sem0, sem1)
    lanes = lax.iota(jnp.int32, L)
    inf_vec = jnp.full((L,), INF, dtype=jnp.float32)
    zero_i = jnp.zeros((L,), jnp.int32)

    for j in range((NB + L) // L):
        blkb[pl.ds(j * L, L)] = zero_i

    def start_row(t, par):
        return pltpu.async_copy(
            d2t_hbm.at[b, q0 + t, :], rowring.at[par], sems[par])

    def wait_row(par):
        pltpu.make_async_copy(
            d2t_hbm.at[b, 0, :], rowring.at[par], sems[par]).wait()

    start_row(0, 0)
    start_row(1, 1)

    def per_target(t, par, sacc):
        mk, _ = plsc.sort_key_val(bmall[t, pl.ds(0, L)], lanes)

        def thr_merge(j, mk):
            ck, _ = plsc.sort_key_val(bmall[t, pl.ds(j * L, L)], lanes)
            nk = jnp.minimum(mk, lax.rev(ck, (0,)))
            sk, _ = plsc.sort_key_val(nk, lanes)
            return sk

        mk = lax.fori_loop(1, NBV, thr_merge, mk)
        thr = jnp.max(mk)

        na = jnp.int32(0)
        for k in range(NBV):
            bmv = bmall[t, pl.ds(k * L, L)]
            msk = bmv <= thr
            ones = jnp.where(msk, 1, 0).astype(jnp.int32)
            cs = plsc.cumsum(ones)
            pos = jnp.maximum(na + cs - 1, 0)
            plsc.store_scatter(blkb, [pos], k * L + lanes, mask=msk)
            na = na + jnp.sum(ones)

        ngrp = (na + L - 1) // L

        def grp_body(gi, cnt):
            blks = blkb[pl.ds(gi * L, L)]
            valid = (gi * L + lanes) < na
            base = blks * BLK
            for k in range(BLK):
                d2v = plsc.load_gather(rowring.at[par], [base + k])
                smsk = (d2v <= thr) & valid
                ones = jnp.where(smsk, 1, 0).astype(jnp.int32)
                cs = plsc.cumsum(ones)
                pos = jnp.maximum(cnt + cs - 1, 0)
                plsc.store_scatter(bufd, [pos], d2v, mask=smsk)
                plsc.store_scatter(bufn, [pos], base + k, mask=smsk)
                cnt = cnt + jnp.sum(ones)
            return cnt

        cnt = lax.fori_loop(0, ngrp, grp_body, jnp.int32(0))

        plsc.store_scatter(bufd, [cnt + lanes], inf_vec)
        plsc.store_scatter(bufn, [cnt + lanes], zero_i)

        bk, bv = plsc.sort_key_val(bufd[pl.ds(0, L)], bufn[pl.ds(0, L)])
        nch = (cnt + L - 1) // L

        def merge_body(j, carry):
            mk2, mv2 = carry
            ck, cv = plsc.sort_key_val(bufd[pl.ds(j * L, L)],
                                       bufn[pl.ds(j * L, L)])
            rk = lax.rev(ck, (0,))
            rv = lax.rev(cv, (0,))
            keep = mk2 <= rk
            nk = jnp.where(keep, mk2, rk)
            nv = jnp.where(keep, mv2, rv)
            sk, sv = plsc.sort_key_val(nk, nv)
            return (sk, sv)

        bk, bv = lax.fori_loop(1, nch, merge_body, (bk, bv))

        xg = plsc.load_gather(xb, [bv])
        w = jnp.float32(1.0) / jnp.maximum(bk, jnp.float32(1e-30))
        qbuf[pl.ds(t * NH, NH)] = xg * w
        return sacc + w

    def pair_body(tp, sacc):
        t0 = tp * 2
        wait_row(0)
        sacc = per_target(t0, 0, sacc)

        @pl.when(t0 + 2 < QPW)
        def _():
            start_row(t0 + 2, 0)

        wait_row(1)
        sacc = per_target(t0 + 1, 1, sacc)

        @pl.when(t0 + 3 < QPW)
        def _():
            start_row(t0 + 3, 1)
        return sacc

    sacc = lax.fori_loop(0, QPW // 2, pair_body,
                         jnp.zeros((L,), jnp.float32))

    sbuf[...] = sacc
    pltpu.sync_copy(qbuf, q_hbm.at[pl.ds((b * T + q0) * NH, QPW * NH)])
    pltpu.sync_copy(sbuf, sp_hbm.at[pl.ds(wid * NH, NH)])


def _sc_topk(xflat, d2t, bm):
    mesh = plsc.VectorSubcoreMesh(core_axis_name="c", subcore_axis_name="s")
    fn = functools.partial(
        pl.kernel, mesh=mesh,
        compiler_params=pltpu.CompilerParams(
            needs_layout_passes=False,
            use_tc_tiling_on_sc=False,
        ),
        out_type=(
            jax.ShapeDtypeStruct((B * T * NH,), jnp.float32),
            jax.ShapeDtypeStruct((NW * NH,), jnp.float32),
        ),
        scratch_types=[
            pltpu.VMEM((N,), jnp.float32),
            pltpu.VMEM((2, N), jnp.float32),
            pltpu.VMEM((QPW, NB), jnp.float32),
            pltpu.VMEM((NB + L,), jnp.int32),
            pltpu.VMEM((SCAP,), jnp.float32),
            pltpu.VMEM((SCAP,), jnp.int32),
            pltpu.VMEM((QPW * NH,), jnp.float32),
            pltpu.VMEM((NH,), jnp.float32),
            pltpu.SemaphoreType.DMA,
            pltpu.SemaphoreType.DMA,
        ],
    )(_sc_body)
    return fn(xflat, d2t, bm)


def _combine_body(q_ref, sp_ref, out_ref):
    sp = sp_ref[...]
    rows = [jnp.sum(sp[8 * bb:8 * bb + 8, :], axis=0, keepdims=True)
            for bb in range(B)]
    s = jnp.concatenate(rows, axis=0)
    arec = jnp.float32(1.0) / s
    r4 = lax.broadcasted_iota(jnp.int32, (T, B), 0)
    c4 = lax.broadcasted_iota(jnp.int32, (T, B), 1)
    i4 = ((r4 // 16) % 4 == c4).astype(jnp.float32)
    p1 = jnp.dot(i4, arec, precision=jax.lax.Precision.HIGHEST)
    rt = lax.broadcasted_iota(jnp.int32, (T, NH), 0)
    ck = lax.broadcasted_iota(jnp.int32, (T, NH), 1)
    k16 = (rt % NH == ck).astype(jnp.float32)
    wcol = jnp.sum(k16 * p1, axis=1, keepdims=True)
    ru = lax.broadcasted_iota(jnp.int32, (64, T), 0)
    ct = lax.broadcasted_iota(jnp.int32, (64, T), 1)
    e = (ct // NH == ru).astype(jnp.float32)
    for bb in range(B):
        z = q_ref[bb] * wcol
        out_ref[bb] = jnp.dot(e, z, precision=jax.lax.Precision.HIGHEST)


def _combine(q, sp):
    return pl.pallas_call(
        _combine_body,
        out_shape=jax.ShapeDtypeStruct((B, 64, NH), jnp.float32),
    )(q, sp)


def kernel(x, coords_rel):
    xflat = x.reshape(B * N)
    d2t, bm = _tc_d2t(coords_rel)
    qflat, spflat = _sc_topk(xflat, d2t, bm)
    q = qflat.reshape(B, T, NH)
    sp = spflat.reshape(NW, NH)
    r = _combine(q, sp)
    return r.transpose(0, 2, 1).reshape(B, T, 1)


# speedup vs baseline: 19.7997x; 1.2627x over previous
import functools

import jax
import jax.numpy as jnp
from jax import lax
from jax.experimental import pallas as pl
from jax.experimental.pallas import tpu as pltpu
from jax.experimental.pallas import tpu_sc as plsc

B = 4
N = 4096
T = 1024
NH = 16
L = 16
NC = 2
NS = 16
NW = NC * NS
QPW = (B * T) // NW
BLK = 16
NB = N // BLK
NBV = NB // L
BT = 128
SCAP = N + L
INF = float("inf")


def _d2t_body(coords_ref, d2t_ref, bm_ref):
    c0 = coords_ref[0, 0]
    c1 = coords_ref[1, 0]
    d2 = c0 * c0 + c1 * c1
    bm = jnp.min(d2.reshape(NB, BLK, BT), axis=1)
    d2t_ref[0] = d2.T.reshape(BT, N // 128, 128)
    bm_ref[0] = bm.T.reshape(BT, NB // 128, 128)


def _tc_d2t(coords):
    return pl.pallas_call(
        _d2t_body,
        grid=(B, T // BT),
        in_specs=[pl.BlockSpec((2, 1, N, BT), lambda b, t: (0, b, 0, t))],
        out_specs=[
            pl.BlockSpec((1, BT, N // 128, 128), lambda b, t: (b, t, 0, 0)),
            pl.BlockSpec((1, BT, NB // 128, 128), lambda b, t: (b, t, 0, 0)),
        ],
        out_shape=[
            jax.ShapeDtypeStruct((B, T, N // 128, 128), jnp.float32),
            jax.ShapeDtypeStruct((B, T, NB // 128, 128), jnp.float32),
        ],
    )(coords)


def _sc_body(x_hbm, d2t_hbm, bm_hbm, q_hbm, sp_hbm,
             xb, rowring, bmall, blkb, bufd, bufn, qbuf, sbuf, sem0, sem1):
    wid = lax.axis_index("s") * NC + lax.axis_index("c")
    b = wid // 8
    q0 = (wid % 8) * QPW

    pltpu.sync_copy(x_hbm.at[pl.ds(b * N, N)], xb)
    pltpu.sync_copy(bm_hbm.at[b, pl.ds(q0, QPW)], bmall)

    sems = (sem0, sem1)
    lanes = lax.iota(jnp.int32, L)
    inf_vec = jnp.full((L,), INF, dtype=jnp.float32)
    zero_i = jnp.zeros((L,), jnp.int32)

    for j in range((NB + L) // L):
        blkb[pl.ds(j * L, L)] = zero_i

    def start_row(t, par):
        return pltpu.async_copy(
            d2t_hbm.at[b, q0 + t], rowring.at[par], sems[par])

    def wait_row(par):
        pltpu.make_async_copy(
            d2t_hbm.at[b, 0], rowring.at[par], sems[par]).wait()

    start_row(0, 0)
    start_row(1, 1)

    def per_target(t, par, sacc):
        mk, _ = plsc.sort_key_val(bmall[t, 0, pl.ds(0, L)], lanes)

        def thr_merge(j, mk):
            ck, _ = plsc.sort_key_val(
                bmall[t, j // 8, pl.ds((j % 8) * L, L)], lanes)
            nk = jnp.minimum(mk, lax.rev(ck, (0,)))
            sk, _ = plsc.sort_key_val(nk, lanes)
            return sk

        mk = lax.fori_loop(1, NBV, thr_merge, mk)
        thr = jnp.max(mk)

        na = jnp.int32(0)
        for k in range(NBV):
            bmv = bmall[t, k // 8, pl.ds((k % 8) * L, L)]
            msk = bmv <= thr
            ones = jnp.where(msk, 1, 0).astype(jnp.int32)
            cs = plsc.cumsum(ones)
            pos = jnp.maximum(na + cs - 1, 0)
            plsc.store_scatter(blkb, [pos], k * L + lanes, mask=msk)
            na = na + jnp.sum(ones)

        ngrp = (na + L - 1) // L

        def grp_body(gi, cnt):
            blks = blkb[pl.ds(gi * L, L)]
            valid = (gi * L + lanes) < na
            base = blks * BLK
            row = blks // 8
            colb = (blks % 8) * BLK
            for k in range(BLK):
                d2v = plsc.load_gather(rowring.at[par], [row, colb + k])
                smsk = (d2v <= thr) & valid
                ones = jnp.where(smsk, 1, 0).astype(jnp.int32)
                cs = plsc.cumsum(ones)
                pos = jnp.maximum(cnt + cs - 1, 0)
                plsc.store_scatter(bufd, [pos], d2v, mask=smsk)
                plsc.store_scatter(bufn, [pos], base + k, mask=smsk)
                cnt = cnt + jnp.sum(ones)
            return cnt

        cnt = lax.fori_loop(0, ngrp, grp_body, jnp.int32(0))

        plsc.store_scatter(bufd, [cnt + lanes], inf_vec)
        plsc.store_scatter(bufn, [cnt + lanes], zero_i)

        bk, bv = plsc.sort_key_val(bufd[pl.ds(0, L)], bufn[pl.ds(0, L)])
        nch = (cnt + L - 1) // L

        def merge_body(j, carry):
            mk2, mv2 = carry
            ck, cv = plsc.sort_key_val(bufd[pl.ds(j * L, L)],
                                       bufn[pl.ds(j * L, L)])
            rk = lax.rev(ck, (0,))
            rv = lax.rev(cv, (0,))
            keep = mk2 <= rk
            nk = jnp.where(keep, mk2, rk)
            nv = jnp.where(keep, mv2, rv)
            sk, sv = plsc.sort_key_val(nk, nv)
            return (sk, sv)

        bk, bv = lax.fori_loop(1, nch, merge_body, (bk, bv))

        xg = plsc.load_gather(xb, [bv])
        w = jnp.float32(1.0) / jnp.maximum(bk, jnp.float32(1e-30))
        qbuf[pl.ds(t * NH, NH)] = xg * w
        return sacc + w

    def pair_body(tp, sacc):
        t0 = tp * 2
        wait_row(0)
        sacc = per_target(t0, 0, sacc)

        @pl.when(t0 + 2 < QPW)
        def _():
            start_row(t0 + 2, 0)

        wait_row(1)
        sacc = per_target(t0 + 1, 1, sacc)

        @pl.when(t0 + 3 < QPW)
        def _():
            start_row(t0 + 3, 1)
        return sacc

    sacc = lax.fori_loop(0, QPW // 2, pair_body,
                         jnp.zeros((L,), jnp.float32))

    sbuf[...] = sacc
    pltpu.sync_copy(qbuf, q_hbm.at[pl.ds((b * T + q0) * NH, QPW * NH)])
    pltpu.sync_copy(sbuf, sp_hbm.at[pl.ds(wid * NH, NH)])


def _sc_topk(xflat, d2t, bm):
    mesh = plsc.VectorSubcoreMesh(core_axis_name="c", subcore_axis_name="s")
    fn = functools.partial(
        pl.kernel, mesh=mesh,
        compiler_params=pltpu.CompilerParams(
            needs_layout_passes=False,
            use_tc_tiling_on_sc=False,
        ),
        out_type=(
            jax.ShapeDtypeStruct((B * T * NH,), jnp.float32),
            jax.ShapeDtypeStruct((NW * NH,), jnp.float32),
        ),
        scratch_types=[
            pltpu.VMEM((N,), jnp.float32),
            pltpu.VMEM((2, N // 128, 128), jnp.float32),
            pltpu.VMEM((QPW, NB // 128, 128), jnp.float32),
            pltpu.VMEM((NB + L,), jnp.int32),
            pltpu.VMEM((SCAP,), jnp.float32),
            pltpu.VMEM((SCAP,), jnp.int32),
            pltpu.VMEM((QPW * NH,), jnp.float32),
            pltpu.VMEM((NH,), jnp.float32),
            pltpu.SemaphoreType.DMA,
            pltpu.SemaphoreType.DMA,
        ],
    )(_sc_body)
    return fn(xflat, d2t, bm)


def _combine_body(q_ref, sp_ref, out_ref):
    sp = sp_ref[...]
    rows = [jnp.sum(sp[8 * bb:8 * bb + 8, :], axis=0, keepdims=True)
            for bb in range(B)]
    s = jnp.concatenate(rows, axis=0)
    arec = jnp.float32(1.0) / s
    r4 = lax.broadcasted_iota(jnp.int32, (T, B), 0)
    c4 = lax.broadcasted_iota(jnp.int32, (T, B), 1)
    i4 = ((r4 // 16) % 4 == c4).astype(jnp.float32)
    p1 = jnp.dot(i4, arec, precision=jax.lax.Precision.HIGHEST)
    rt = lax.broadcasted_iota(jnp.int32, (T, NH), 0)
    ck = lax.broadcasted_iota(jnp.int32, (T, NH), 1)
    k16 = (rt % NH == ck).astype(jnp.float32)
    wcol = jnp.sum(k16 * p1, axis=1, keepdims=True)
    ru = lax.broadcasted_iota(jnp.int32, (64, T), 0)
    ct = lax.broadcasted_iota(jnp.int32, (64, T), 1)
    e = (ct // NH == ru).astype(jnp.float32)
    for bb in range(B):
        z = q_ref[bb] * wcol
        out_ref[bb] = jnp.dot(e, z, precision=jax.lax.Precision.HIGHEST)


def _combine(q, sp):
    return pl.pallas_call(
        _combine_body,
        out_shape=jax.ShapeDtypeStruct((B, 64, NH), jnp.float32),
    )(q, sp)


def kernel(x, coords_rel):
    xflat = x.reshape(B * N)
    d2t, bm = _tc_d2t(coords_rel)
    qflat, spflat = _sc_topk(xflat, d2t, bm)
    q = qflat.reshape(B, T, NH)
    sp = spflat.reshape(NW, NH)
    r = _combine(q, sp)
    return r.transpose(0, 2, 1).reshape(B, T, 1)


# speedup vs baseline: 20.8312x; 1.0521x over previous
import functools

import jax
import jax.numpy as jnp
from jax import lax
from jax.experimental import pallas as pl
from jax.experimental.pallas import tpu as pltpu
from jax.experimental.pallas import tpu_sc as plsc

B = 4
N = 4096
T = 1024
NH = 16
L = 16
NC = 2
NS = 16
NW = NC * NS
QPW = (2 * T) // NW
BLK = 16
NB = N // BLK
NBV = NB // L
BT = 128
SCAP = N + L
INF = float("inf")


def _d2t_body(coords_ref, d2t_ref, bm_ref):
    c0 = coords_ref[0, 0]
    c1 = coords_ref[1, 0]
    d2 = c0 * c0 + c1 * c1
    bm = jnp.min(d2.reshape(NB, BLK, BT), axis=1)
    d2t_ref[0] = d2.T.reshape(BT, N // 128, 128)
    bm_ref[0] = bm.T.reshape(BT, NB // 128, 128)


def _tc_d2t(coords, b0):
    return pl.pallas_call(
        _d2t_body,
        grid=(2, T // BT),
        in_specs=[pl.BlockSpec((2, 1, N, BT),
                               lambda b, t: (0, b0 + b, 0, t))],
        out_specs=[
            pl.BlockSpec((1, BT, N // 128, 128), lambda b, t: (b, t, 0, 0)),
            pl.BlockSpec((1, BT, NB // 128, 128), lambda b, t: (b, t, 0, 0)),
        ],
        out_shape=[
            jax.ShapeDtypeStruct((2, T, N // 128, 128), jnp.float32),
            jax.ShapeDtypeStruct((2, T, NB // 128, 128), jnp.float32),
        ],
    )(coords)


def _sc_body(b0, x_hbm, d2t_hbm, bm_hbm, q_hbm, sp_hbm,
             xb, rowring, bmall, blkb, bufd, bufn, qbuf, sbuf, sem0, sem1):
    wid = lax.axis_index("s") * NC + lax.axis_index("c")
    b = wid // 16
    q0 = (wid % 16) * QPW

    pltpu.sync_copy(x_hbm.at[pl.ds((b0 + b) * N, N)], xb)
    pltpu.sync_copy(bm_hbm.at[b, pl.ds(q0, QPW)], bmall)

    sems = (sem0, sem1)
    lanes = lax.iota(jnp.int32, L)
    inf_vec = jnp.full((L,), INF, dtype=jnp.float32)
    zero_i = jnp.zeros((L,), jnp.int32)

    for j in range((NB + L) // L):
        blkb[pl.ds(j * L, L)] = zero_i

    def start_row(t, par):
        return pltpu.async_copy(
            d2t_hbm.at[b, q0 + t], rowring.at[par], sems[par])

    def wait_row(par):
        pltpu.make_async_copy(
            d2t_hbm.at[b, 0], rowring.at[par], sems[par]).wait()

    start_row(0, 0)
    start_row(1, 1)

    def per_target(t, par, sacc):
        mk, _ = plsc.sort_key_val(bmall[t, 0, pl.ds(0, L)], lanes)

        def thr_merge(j, mk):
            ck, _ = plsc.sort_key_val(
                bmall[t, j // 8, pl.ds((j % 8) * L, L)], lanes)
            nk = jnp.minimum(mk, lax.rev(ck, (0,)))
            sk, _ = plsc.sort_key_val(nk, lanes)
            return sk

        mk = lax.fori_loop(1, NBV, thr_merge, mk)
        thr = jnp.max(mk)

        na = jnp.int32(0)
        for k in range(NBV):
            bmv = bmall[t, k // 8, pl.ds((k % 8) * L, L)]
            msk = bmv <= thr
            ones = jnp.where(msk, 1, 0).astype(jnp.int32)
            cs = plsc.cumsum(ones)
            pos = jnp.maximum(na + cs - 1, 0)
            plsc.store_scatter(blkb, [pos], k * L + lanes, mask=msk)
            na = na + jnp.sum(ones)

        ngrp = (na + L - 1) // L

        def grp_body(gi, cnt):
            blks = blkb[pl.ds(gi * L, L)]
            valid = (gi * L + lanes) < na
            base = blks * BLK
            row = blks // 8
            colb = (blks % 8) * BLK
            for k in range(BLK):
                d2v = plsc.load_gather(rowring.at[par], [row, colb + k])
                smsk = (d2v <= thr) & valid
                ones = jnp.where(smsk, 1, 0).astype(jnp.int32)
                cs = plsc.cumsum(ones)
                pos = jnp.maximum(cnt + cs - 1, 0)
                plsc.store_scatter(bufd, [pos], d2v, mask=smsk)
                plsc.store_scatter(bufn, [pos], base + k, mask=smsk)
                cnt = cnt + jnp.sum(ones)
            return cnt

        cnt = lax.fori_loop(0, ngrp, grp_body, jnp.int32(0))

        plsc.store_scatter(bufd, [cnt + lanes], inf_vec)
        plsc.store_scatter(bufn, [cnt + lanes], zero_i)

        bk, bv = plsc.sort_key_val(bufd[pl.ds(0, L)], bufn[pl.ds(0, L)])
        nch = (cnt + L - 1) // L

        def merge_body(j, carry):
            mk2, mv2 = carry
            ck, cv = plsc.sort_key_val(bufd[pl.ds(j * L, L)],
                                       bufn[pl.ds(j * L, L)])
            rk = lax.rev(ck, (0,))
            rv = lax.rev(cv, (0,))
            keep = mk2 <= rk
            nk = jnp.where(keep, mk2, rk)
            nv = jnp.where(keep, mv2, rv)
            sk, sv = plsc.sort_key_val(nk, nv)
            return (sk, sv)

        bk, bv = lax.fori_loop(1, nch, merge_body, (bk, bv))

        xg = plsc.load_gather(xb, [bv])
        w = jnp.float32(1.0) / jnp.maximum(bk, jnp.float32(1e-30))
        qbuf[pl.ds(t * NH, NH)] = xg * w
        return sacc + w

    def pair_body(tp, sacc):
        t0 = tp * 2
        wait_row(0)
        sacc = per_target(t0, 0, sacc)

        @pl.when(t0 + 2 < QPW)
        def _():
            start_row(t0 + 2, 0)

        wait_row(1)
        sacc = per_target(t0 + 1, 1, sacc)

        @pl.when(t0 + 3 < QPW)
        def _():
            start_row(t0 + 3, 1)
        return sacc

    sacc = lax.fori_loop(0, QPW // 2, pair_body,
                         jnp.zeros((L,), jnp.float32))

    sbuf[...] = sacc
    pltpu.sync_copy(qbuf, q_hbm.at[pl.ds((b * T + q0) * NH, QPW * NH)])
    pltpu.sync_copy(sbuf, sp_hbm.at[pl.ds(wid * NH, NH)])


def _sc_topk(xflat, d2t, bm, b0):
    mesh = plsc.VectorSubcoreMesh(core_axis_name="c", subcore_axis_name="s")
    fn = functools.partial(
        pl.kernel, mesh=mesh,
        compiler_params=pltpu.CompilerParams(
            needs_layout_passes=False,
            use_tc_tiling_on_sc=False,
        ),
        out_type=(
            jax.ShapeDtypeStruct((2 * T * NH,), jnp.float32),
            jax.ShapeDtypeStruct((NW * NH,), jnp.float32),
        ),
        scratch_types=[
            pltpu.VMEM((N,), jnp.float32),
            pltpu.VMEM((2, N // 128, 128), jnp.float32),
            pltpu.VMEM((QPW, NB // 128, 128), jnp.float32),
            pltpu.VMEM((NB + L,), jnp.int32),
            pltpu.VMEM((SCAP,), jnp.float32),
            pltpu.VMEM((SCAP,), jnp.int32),
            pltpu.VMEM((QPW * NH,), jnp.float32),
            pltpu.VMEM((NH,), jnp.float32),
            pltpu.SemaphoreType.DMA,
            pltpu.SemaphoreType.DMA,
        ],
    )(functools.partial(_sc_body, b0))
    return fn(xflat, d2t, bm)


def _combine_body(q_ref, sp_ref, out_ref):
    sp = sp_ref[...]
    rows = [jnp.sum(sp[16 * bb:16 * bb + 16, :], axis=0, keepdims=True)
            for bb in range(B)]
    s = jnp.concatenate(rows, axis=0)
    arec = jnp.float32(1.0) / s
    r4 = lax.broadcasted_iota(jnp.int32, (T, B), 0)
    c4 = lax.broadcasted_iota(jnp.int32, (T, B), 1)
    i4 = ((r4 // 16) % 4 == c4).astype(jnp.float32)
    p1 = jnp.dot(i4, arec, precision=jax.lax.Precision.HIGHEST)
    rt = lax.broadcasted_iota(jnp.int32, (T, NH), 0)
    ck = lax.broadcasted_iota(jnp.int32, (T, NH), 1)
    k16 = (rt % NH == ck).astype(jnp.float32)
    wcol = jnp.sum(k16 * p1, axis=1, keepdims=True)
    ru = lax.broadcasted_iota(jnp.int32, (64, T), 0)
    ct = lax.broadcasted_iota(jnp.int32, (64, T), 1)
    e = (ct // NH == ru).astype(jnp.float32)
    for bb in range(B):
        z = q_ref[bb] * wcol
        out_ref[bb] = jnp.dot(e, z, precision=jax.lax.Precision.HIGHEST)


def _combine(q, sp):
    return pl.pallas_call(
        _combine_body,
        out_shape=jax.ShapeDtypeStruct((B, 64, NH), jnp.float32),
    )(q, sp)


def kernel(x, coords_rel):
    xflat = x.reshape(B * N)
    d2t01, bm01 = _tc_d2t(coords_rel, 0)
    q01, sp01 = _sc_topk(xflat, d2t01, bm01, 0)
    d2t23, bm23 = _tc_d2t(coords_rel, 2)
    q23, sp23 = _sc_topk(xflat, d2t23, bm23, 2)
    q = jnp.concatenate([q01.reshape(2, T, NH), q23.reshape(2, T, NH)])
    sp = jnp.concatenate([sp01.reshape(NW, NH), sp23.reshape(NW, NH)])
    r = _combine(q, sp)
    return r.transpose(0, 2, 1).reshape(B, T, 1)
